# baseline (speedup 1.0000x reference)
import functools

import jax
import jax.numpy as jnp
from jax import lax
from jax.experimental import pallas as pl
from jax.experimental.pallas import tpu as pltpu
from jax.experimental.pallas import tpu_sc as plsc

N_NODES = 10000
N_EDGES = 320000
D = 128
NC = 2
NS = 16
NW = NC * NS
EPW = N_EDGES // NW
K = 96
EPP = 10176
PAD = EPP - EPW
NB = EPP // K
ACC_N = 10240
RPT = ACC_N // NS
CH = ACC_N // D


def _sc_body(x_hbm, er_hbm, out_hbm, cnt_hbm,
             s0, s1, s2, d0, d1, d2, rows0, rows1, rows2, cnt, acc,
             semg0, semg1, semg2, sems,
             semi0, semi1, semi2, semd0, semd1, semd2):
    c = lax.axis_index("c")
    s = lax.axis_index("s")
    wid = c * NS + s

    zero16 = jnp.zeros((16,), jnp.float32)
    ones16 = jnp.ones((16,), jnp.float32)

    srcs = ((s0, semi0), (s1, semi1), (s2, semi2))
    dsts = ((d0, semd0), (d1, semd1), (d2, semd2))
    rows = ((rows0, semg0), (rows1, semg1), (rows2, semg2))

    def zcnt(i, carry):
        cnt[pl.ds(i * 16, 16)] = zero16
        return carry

    lax.fori_loop(0, ACC_N // 16, zcnt, 0)

    def zrow(r, carry):
        for i in range(D // 16):
            rows0[r, pl.ds(i * 16, 16)] = zero16
        return carry

    lax.fori_loop(0, K, zrow, 0)

    for i in range(6):
        pltpu.sync_copy(rows0, acc.at[pl.ds(s * RPT + i * K, K), :])
    pltpu.sync_copy(rows0.at[pl.ds(0, RPT - 6 * K), :],
                    acc.at[pl.ds(s * RPT + 6 * K, RPT - 6 * K), :])
    plsc.subcore_barrier()

    def src_load(j, q):
        buf, sem = srcs[q]
        pltpu.async_copy(er_hbm.at[pl.ds(wid * EPP + j * K, K)], buf, sem)

    def src_wait(j, q):
        buf, sem = srcs[q]
        pltpu.make_async_copy(
            er_hbm.at[pl.ds(wid * EPP + j * K, K)], buf, sem).wait()

    def dst_load(j, q):
        buf, sem = dsts[q]
        pltpu.async_copy(
            er_hbm.at[pl.ds((NW + wid) * EPP + j * K, K)], buf, sem)

    def dst_wait(j, q):
        buf, sem = dsts[q]
        pltpu.make_async_copy(
            er_hbm.at[pl.ds((NW + wid) * EPP + j * K, K)], buf, sem).wait()

    def gather_start(q):
        buf, sem = rows[q]
        pltpu.async_copy(x_hbm.at[srcs[q][0]], buf, sem)

    def gather_wait(q):
        buf, sem = rows[q]
        pltpu.make_async_copy(x_hbm.at[srcs[q][0]], buf, sem).wait()

    def scatter_start(q):
        pltpu.async_copy(rows[q][0], acc.at[dsts[q][0]], sems, add=True)

    def scatter_wait(q):
        pltpu.make_async_copy(rows[q][0], acc.at[dsts[q][0]], sems).wait()

    def histogram(q):
        d = dsts[q][0]
        for g in range(K // 16):
            idx16 = d[pl.ds(g * 16, 16)]
            plsc.addupdate_scatter(cnt, [idx16], ones16)

    for j in range(3):
        src_load(j, j)
    for j in range(2):
        dst_load(j, j)
    src_wait(0, 0)
    gather_start(0)
    src_wait(1, 1)
    gather_start(1)

    def body(j3, carry):
        for u in range(3):
            j = j3 * 3 + u
            q = u
            qm = (u + 2) % 3
            gather_wait(q)

            @pl.when(j >= 1)
            def _():
                scatter_wait(qm)

            @pl.when(j + 2 < NB)
            def _():
                dst_load(j + 2, qm)
                src_wait(j + 2, qm)
                gather_start(qm)

            @pl.when(j + 3 < NB)
            def _():
                src_load(j + 3, q)

            dst_wait(j, q)
            histogram(q)
            scatter_start(q)

        return carry

    lax.fori_loop(0, NB // 3, body, 0)

    for j in range(3 * (NB // 3), NB):
        q = j % 3
        gather_wait(q)
        scatter_wait((q + 2) % 3)
        dst_wait(j, q)
        histogram(q)
        scatter_start(q)
    scatter_wait((NB - 1) % 3)
    plsc.subcore_barrier()

    pltpu.sync_copy(cnt, cnt_hbm.at[pl.ds(wid * ACC_N, ACC_N)])
    for i in range(6):
        r0 = s * RPT + i * K
        pltpu.sync_copy(acc.at[pl.ds(r0, K), :],
                        out_hbm.at[c, pl.ds(r0, K), :])
    r0 = s * RPT + 6 * K
    pltpu.sync_copy(acc.at[pl.ds(r0, RPT - 6 * K), :],
                    out_hbm.at[c, pl.ds(r0, RPT - 6 * K), :])


_sc_gather_scatter = functools.partial(
    pl.kernel,
    out_type=(jax.ShapeDtypeStruct((NC, ACC_N, D), jnp.float32),
              jax.ShapeDtypeStruct((NW * ACC_N,), jnp.float32)),
    mesh=plsc.VectorSubcoreMesh(core_axis_name="c", subcore_axis_name="s"),
    compiler_params=pltpu.CompilerParams(needs_layout_passes=False),
    scratch_types=[
        pltpu.VMEM((K,), jnp.int32), pltpu.VMEM((K,), jnp.int32),
        pltpu.VMEM((K,), jnp.int32),
        pltpu.VMEM((K,), jnp.int32), pltpu.VMEM((K,), jnp.int32),
        pltpu.VMEM((K,), jnp.int32),
        pltpu.VMEM((K, D), jnp.float32), pltpu.VMEM((K, D), jnp.float32),
        pltpu.VMEM((K, D), jnp.float32),
        pltpu.VMEM((ACC_N,), jnp.float32),
        pltpu.VMEM_SHARED((ACC_N, D), jnp.float32),
        pltpu.SemaphoreType.DMA, pltpu.SemaphoreType.DMA,
        pltpu.SemaphoreType.DMA, pltpu.SemaphoreType.DMA,
        pltpu.SemaphoreType.DMA, pltpu.SemaphoreType.DMA,
        pltpu.SemaphoreType.DMA, pltpu.SemaphoreType.DMA,
        pltpu.SemaphoreType.DMA, pltpu.SemaphoreType.DMA,
    ],
)(_sc_body)


def _tc_pre_body(x_ref, bl_ref, wr_ref, o_ref):
    o_ref[...] = lax.dot_general(
        x_ref[...], wr_ref[...], (((1,), (1,)), ((), ())),
        preferred_element_type=jnp.float32) + bl_ref[...]


_tc_pre = pl.pallas_call(
    _tc_pre_body,
    out_shape=jax.ShapeDtypeStruct((N_NODES, D), jnp.float32),
)


def _tc_body(part_ref, cnt_ref, x_ref, xr_ref, wl_ref, o_ref):
    p = part_ref[0] + part_ref[1]
    gsum = p[:N_NODES]
    cnt8 = jnp.sum(cnt_ref[...], axis=0)
    rsel = (lax.broadcasted_iota(jnp.int32, (ACC_N, CH), 0) >> 7
            == lax.broadcasted_iota(jnp.int32, (ACC_N, CH), 1))
    lsel = ((lax.broadcasted_iota(jnp.int32, (ACC_N, D), 0) & (D - 1))
            == lax.broadcasted_iota(jnp.int32, (ACC_N, D), 1))
    cnt_rows = lax.dot_general(rsel.astype(jnp.float32), cnt8,
                               (((1,), (0,)), ((), ())),
                               preferred_element_type=jnp.float32)
    cnt = jnp.sum(cnt_rows * lsel.astype(jnp.float32), axis=1,
                  keepdims=True)[:N_NODES]
    xb = x_ref[...]
    agg = gsum / jnp.maximum(cnt, 1.0) - xb * (cnt > 0.0).astype(jnp.float32)
    o_ref[...] = (
        lax.dot_general(agg, wl_ref[...], (((1,), (1,)), ((), ())),
                        preferred_element_type=jnp.float32)
        + xr_ref[...]
    )


_tc_finish = pl.pallas_call(
    _tc_body,
    out_shape=jax.ShapeDtypeStruct((N_NODES, D), jnp.float32),
)


def kernel(x, edge_index, W_l, b_l, W_r):
    src2 = jnp.pad(edge_index[0].reshape(NW, EPW), ((0, 0), (0, PAD)))
    dst2 = jnp.pad(edge_index[1].reshape(NW, EPW), ((0, 0), (0, PAD)),
                   constant_values=ACC_N - 1)
    er = jnp.concatenate([src2.reshape(-1), dst2.reshape(-1)])
    part, cnts = _sc_gather_scatter(x, er)
    xr = _tc_pre(x, b_l.reshape(1, D), W_r)
    return _tc_finish(part, cnts.reshape(NW, CH, D), x, xr, W_l)


# speedup vs baseline: 1.0004x; 1.0004x over previous
import functools

import jax
import jax.numpy as jnp
from jax import lax
from jax.experimental import pallas as pl
from jax.experimental.pallas import tpu as pltpu
from jax.experimental.pallas import tpu_sc as plsc

N_NODES = 10000
N_EDGES = 320000
D = 128
NC = 2
NS = 16
NW = NC * NS
EPW = N_EDGES // NW
K = 96
EPP = 10176
PAD = EPP - EPW
NB = EPP // K
ACC_N = 10240
RPT = ACC_N // NS
CH = ACC_N // D


def _sc_body(x_hbm, er_hbm, out_hbm, cnt_hbm,
             s0, s1, s2, d0, d1, d2, rows0, rows1, rows2, cnt, acc,
             semg0, semg1, semg2, sems,
             semi0, semi1, semi2, semd0, semd1, semd2):
    c = lax.axis_index("c")
    s = lax.axis_index("s")
    wid = c * NS + s

    zero16 = jnp.zeros((16,), jnp.float32)
    ones16 = jnp.ones((16,), jnp.float32)

    srcs = ((s0, semi0), (s1, semi1), (s2, semi2))
    dsts = ((d0, semd0), (d1, semd1), (d2, semd2))
    rows = ((rows0, semg0), (rows1, semg1), (rows2, semg2))

    def zcnt(i, carry):
        cnt[pl.ds(i * 16, 16)] = zero16
        return carry

    lax.fori_loop(0, ACC_N // 16, zcnt, 0)

    def zrow(r, carry):
        for i in range(D // 16):
            rows0[r, pl.ds(i * 16, 16)] = zero16
        return carry

    lax.fori_loop(0, K, zrow, 0)

    for i in range(6):
        pltpu.sync_copy(rows0, acc.at[pl.ds(s * RPT + i * K, K), :])
    pltpu.sync_copy(rows0.at[pl.ds(0, RPT - 6 * K), :],
                    acc.at[pl.ds(s * RPT + 6 * K, RPT - 6 * K), :])
    plsc.subcore_barrier()

    def src_load(j, q):
        buf, sem = srcs[q]
        pltpu.async_copy(er_hbm.at[pl.ds(wid * EPP + j * K, K)], buf, sem)

    def src_wait(j, q):
        buf, sem = srcs[q]
        pltpu.make_async_copy(
            er_hbm.at[pl.ds(wid * EPP + j * K, K)], buf, sem).wait()

    def dst_load(j, q):
        buf, sem = dsts[q]
        pltpu.async_copy(
            er_hbm.at[pl.ds((NW + wid) * EPP + j * K, K)], buf, sem)

    def dst_wait(j, q):
        buf, sem = dsts[q]
        pltpu.make_async_copy(
            er_hbm.at[pl.ds((NW + wid) * EPP + j * K, K)], buf, sem).wait()

    def gather_start(q):
        buf, sem = rows[q]
        pltpu.async_copy(x_hbm.at[srcs[q][0]], buf, sem)

    def gather_wait(q):
        buf, sem = rows[q]
        pltpu.make_async_copy(x_hbm.at[srcs[q][0]], buf, sem).wait()

    def scatter_start(q):
        pltpu.async_copy(rows[q][0], acc.at[dsts[q][0]], sems, add=True)

    def scatter_wait(q):
        pltpu.make_async_copy(rows[q][0], acc.at[dsts[q][0]], sems).wait()

    def histogram(q):
        d = dsts[q][0]
        for g in range(K // 16):
            idx16 = d[pl.ds(g * 16, 16)]
            plsc.addupdate_scatter(cnt, [idx16], ones16)

    for j in range(3):
        src_load(j, j)
    for j in range(2):
        dst_load(j, j)
    src_wait(0, 0)
    gather_start(0)
    src_wait(1, 1)
    gather_start(1)

    def body(j3, carry):
        for u in range(3):
            j = j3 * 3 + u
            q = u
            qm = (u + 2) % 3
            gather_wait(q)

            @pl.when(j >= 1)
            def _():
                scatter_wait(qm)

            @pl.when(j + 2 < NB)
            def _():
                dst_load(j + 2, qm)
                src_wait(j + 2, qm)
                gather_start(qm)

            @pl.when(j + 3 < NB)
            def _():
                src_load(j + 3, q)

            dst_wait(j, q)
            histogram(q)
            scatter_start(q)

        return carry

    lax.fori_loop(0, NB // 3, body, 0)

    for j in range(3 * (NB // 3), NB):
        q = j % 3
        gather_wait(q)
        scatter_wait((q + 2) % 3)
        dst_wait(j, q)
        histogram(q)
        scatter_start(q)
    scatter_wait((NB - 1) % 3)
    plsc.subcore_barrier()

    pltpu.sync_copy(cnt, cnt_hbm.at[pl.ds(wid * ACC_N, ACC_N)])
    for i in range(6):
        r0 = s * RPT + i * K
        pltpu.sync_copy(acc.at[pl.ds(r0, K), :],
                        out_hbm.at[c, pl.ds(r0, K), :])
    r0 = s * RPT + 6 * K
    pltpu.sync_copy(acc.at[pl.ds(r0, RPT - 6 * K), :],
                    out_hbm.at[c, pl.ds(r0, RPT - 6 * K), :])


_sc_gather_scatter = functools.partial(
    pl.kernel,
    out_type=(jax.ShapeDtypeStruct((NC, ACC_N, D), jnp.float32),
              jax.ShapeDtypeStruct((NW * ACC_N,), jnp.float32)),
    mesh=plsc.VectorSubcoreMesh(core_axis_name="c", subcore_axis_name="s"),
    compiler_params=pltpu.CompilerParams(needs_layout_passes=False),
    scratch_types=[
        pltpu.VMEM((K,), jnp.int32), pltpu.VMEM((K,), jnp.int32),
        pltpu.VMEM((K,), jnp.int32),
        pltpu.VMEM((K,), jnp.int32), pltpu.VMEM((K,), jnp.int32),
        pltpu.VMEM((K,), jnp.int32),
        pltpu.VMEM((K, D), jnp.float32), pltpu.VMEM((K, D), jnp.float32),
        pltpu.VMEM((K, D), jnp.float32),
        pltpu.VMEM((ACC_N,), jnp.float32),
        pltpu.VMEM_SHARED((ACC_N, D), jnp.float32),
        pltpu.SemaphoreType.DMA, pltpu.SemaphoreType.DMA,
        pltpu.SemaphoreType.DMA, pltpu.SemaphoreType.DMA,
        pltpu.SemaphoreType.DMA, pltpu.SemaphoreType.DMA,
        pltpu.SemaphoreType.DMA, pltpu.SemaphoreType.DMA,
        pltpu.SemaphoreType.DMA, pltpu.SemaphoreType.DMA,
    ],
)(_sc_body)


def _tc_pre_body(x_ref, bl_ref, wr_ref, o_ref):
    o_ref[...] = lax.dot_general(
        x_ref[...], wr_ref[...], (((1,), (1,)), ((), ())),
        preferred_element_type=jnp.float32) + bl_ref[...]


_tc_pre = pl.pallas_call(
    _tc_pre_body,
    out_shape=jax.ShapeDtypeStruct((N_NODES, D), jnp.float32),
)


def _tc_body(part_ref, cnt_ref, x_ref, xr_ref, wl_ref, o_ref):
    p = part_ref[0] + part_ref[1]
    gsum = p[:N_NODES]
    cnt8 = jnp.sum(cnt_ref[...], axis=0)
    rsel = (lax.broadcasted_iota(jnp.int32, (ACC_N, CH), 0) >> 7
            == lax.broadcasted_iota(jnp.int32, (ACC_N, CH), 1))
    lsel = ((lax.broadcasted_iota(jnp.int32, (ACC_N, D), 0) & (D - 1))
            == lax.broadcasted_iota(jnp.int32, (ACC_N, D), 1))
    cnt_rows = lax.dot_general(rsel.astype(jnp.float32), cnt8,
                               (((1,), (0,)), ((), ())),
                               preferred_element_type=jnp.float32)
    cnt = jnp.sum(cnt_rows * lsel.astype(jnp.float32), axis=1,
                  keepdims=True)[:N_NODES]
    xb = x_ref[...]
    agg = gsum / jnp.maximum(cnt, 1.0) - xb * (cnt > 0.0).astype(jnp.float32)
    o_ref[...] = (
        lax.dot_general(agg, wl_ref[...], (((1,), (1,)), ((), ())),
                        preferred_element_type=jnp.float32)
        + xr_ref[...]
    )


_tc_finish = pl.pallas_call(
    _tc_body,
    out_shape=jax.ShapeDtypeStruct((N_NODES, D), jnp.float32),
)


def kernel(x, edge_index, W_l, b_l, W_r):
    src2 = jnp.pad(edge_index[0].reshape(NW, EPW), ((0, 0), (0, PAD)))
    pad_ids = (N_NODES
               + jnp.arange(NW * PAD, dtype=jnp.int32) % (ACC_N - N_NODES)
               ).reshape(NW, PAD)
    dst2 = jnp.concatenate([edge_index[1].reshape(NW, EPW), pad_ids], axis=1)
    er = jnp.concatenate([src2.reshape(-1), dst2.reshape(-1)])
    part, cnts = _sc_gather_scatter(x, er)
    xr = _tc_pre(x, b_l.reshape(1, D), W_r)
    return _tc_finish(part, cnts.reshape(NW, CH, D), x, xr, W_l)


# speedup vs baseline: 2.6973x; 2.6963x over previous
import functools

import jax
import jax.numpy as jnp
from jax import lax
from jax.experimental import pallas as pl
from jax.experimental.pallas import tpu as pltpu
from jax.experimental.pallas import tpu_sc as plsc

N_NODES = 10000
N_EDGES = 320000
D = 128
NC = 2
NS = 16
NW = NC * NS
EPW = N_EDGES // NW
K = 80
NB = EPW // K
ACC_N = 10240
RPT = ACC_N // NS
CH = ACC_N // D


def _sc_body(x_hbm, er_hbm, out_hbm, cnt_hbm,
             s0, s1, s2, d0, d1, d2, rows0, rows1, rows2, cnt, acc,
             semg0, semg1, semg2, sems,
             semi0, semi1, semi2, semd0, semd1, semd2):
    c = lax.axis_index("c")
    s = lax.axis_index("s")
    wid = c * NS + s

    zero16 = jnp.zeros((16,), jnp.float32)
    ones16 = jnp.ones((16,), jnp.float32)

    srcs = ((s0, semi0), (s1, semi1), (s2, semi2))
    dsts = ((d0, semd0), (d1, semd1), (d2, semd2))
    rows = ((rows0, semg0), (rows1, semg1), (rows2, semg2))

    def zcnt(i, carry):
        cnt[pl.ds(i * 16, 16)] = zero16
        return carry

    lax.fori_loop(0, ACC_N // 16, zcnt, 0)

    def zrow(r, carry):
        for i in range(D // 16):
            rows0[r, pl.ds(i * 16, 16)] = zero16
        return carry

    lax.fori_loop(0, K, zrow, 0)

    for i in range(RPT // K):
        pltpu.async_copy(rows0, acc.at[pl.ds(s * RPT + i * K, K), :], semi0)
    for i in range(RPT // K):
        pltpu.make_async_copy(
            rows0, acc.at[pl.ds(s * RPT + i * K, K), :], semi0).wait()
    plsc.subcore_barrier()

    def src_load(j, q):
        buf, sem = srcs[q]
        pltpu.async_copy(er_hbm.at[pl.ds(wid * EPW + j * K, K)], buf, sem)

    def src_wait(j, q):
        buf, sem = srcs[q]
        pltpu.make_async_copy(
            er_hbm.at[pl.ds(wid * EPW + j * K, K)], buf, sem).wait()

    def dst_load(j, q):
        buf, sem = dsts[q]
        pltpu.async_copy(
            er_hbm.at[pl.ds((NW + wid) * EPW + j * K, K)], buf, sem)

    def dst_wait(j, q):
        buf, sem = dsts[q]
        pltpu.make_async_copy(
            er_hbm.at[pl.ds((NW + wid) * EPW + j * K, K)], buf, sem).wait()

    def gather_start(q):
        buf, sem = rows[q]
        pltpu.async_copy(x_hbm.at[srcs[q][0]], buf, sem)

    def gather_wait(q):
        buf, sem = rows[q]
        pltpu.make_async_copy(x_hbm.at[srcs[q][0]], buf, sem).wait()

    def scatter_start(q):
        pltpu.async_copy(rows[q][0], acc.at[dsts[q][0]], sems, add=True)

    def scatter_wait(q):
        pltpu.make_async_copy(rows[q][0], acc.at[dsts[q][0]], sems).wait()

    def histogram(q):
        d = dsts[q][0]
        for g in range(K // 16):
            idx16 = d[pl.ds(g * 16, 16)]
            plsc.addupdate_scatter(cnt, [idx16], ones16)

    for j in range(3):
        src_load(j, j)
    for j in range(2):
        dst_load(j, j)
    src_wait(0, 0)
    gather_start(0)
    src_wait(1, 1)
    gather_start(1)

    def body(j3, carry):
        for u in range(3):
            j = j3 * 3 + u
            q = u
            qm = (u + 2) % 3
            gather_wait(q)

            @pl.when(j >= 1)
            def _():
                scatter_wait(qm)

            @pl.when(j + 2 < NB)
            def _():
                dst_load(j + 2, qm)
                src_wait(j + 2, qm)
                gather_start(qm)

            @pl.when(j + 3 < NB)
            def _():
                src_load(j + 3, q)

            dst_wait(j, q)
            histogram(q)
            scatter_start(q)

        return carry

    lax.fori_loop(0, NB // 3, body, 0)

    for j, q in ((NB - 2, 0), (NB - 1, 1)):
        gather_wait(q)
        scatter_wait((q + 2) % 3)
        dst_wait(j, q)
        histogram(q)
        scatter_start(q)
    scatter_wait(1)
    plsc.subcore_barrier()

    pltpu.async_copy(cnt, cnt_hbm.at[pl.ds(wid * ACC_N, ACC_N)], semi0)
    for i in range(RPT // K):
        r0 = s * RPT + i * K
        pltpu.async_copy(acc.at[pl.ds(r0, K), :],
                         out_hbm.at[c, pl.ds(r0, K), :], semi0)
    pltpu.make_async_copy(cnt, cnt_hbm.at[pl.ds(wid * ACC_N, ACC_N)],
                          semi0).wait()
    for i in range(RPT // K):
        r0 = s * RPT + i * K
        pltpu.make_async_copy(acc.at[pl.ds(r0, K), :],
                              out_hbm.at[c, pl.ds(r0, K), :], semi0).wait()


_sc_gather_scatter = functools.partial(
    pl.kernel,
    out_type=(jax.ShapeDtypeStruct((NC, ACC_N, D), jnp.float32),
              jax.ShapeDtypeStruct((NW * ACC_N,), jnp.float32)),
    mesh=plsc.VectorSubcoreMesh(core_axis_name="c", subcore_axis_name="s"),
    compiler_params=pltpu.CompilerParams(needs_layout_passes=False),
    scratch_types=[
        pltpu.VMEM((K,), jnp.int32), pltpu.VMEM((K,), jnp.int32),
        pltpu.VMEM((K,), jnp.int32),
        pltpu.VMEM((K,), jnp.int32), pltpu.VMEM((K,), jnp.int32),
        pltpu.VMEM((K,), jnp.int32),
        pltpu.VMEM((K, D), jnp.float32), pltpu.VMEM((K, D), jnp.float32),
        pltpu.VMEM((K, D), jnp.float32),
        pltpu.VMEM((ACC_N,), jnp.float32),
        pltpu.VMEM_SHARED((ACC_N, D), jnp.float32),
        pltpu.SemaphoreType.DMA, pltpu.SemaphoreType.DMA,
        pltpu.SemaphoreType.DMA, pltpu.SemaphoreType.DMA,
        pltpu.SemaphoreType.DMA, pltpu.SemaphoreType.DMA,
        pltpu.SemaphoreType.DMA, pltpu.SemaphoreType.DMA,
        pltpu.SemaphoreType.DMA, pltpu.SemaphoreType.DMA,
    ],
)(_sc_body)


def _tc_pre_body(x_ref, bl_ref, wr_ref, o_ref):
    o_ref[...] = lax.dot_general(
        x_ref[...], wr_ref[...], (((1,), (1,)), ((), ())),
        preferred_element_type=jnp.float32) + bl_ref[...]


_tc_pre = pl.pallas_call(
    _tc_pre_body,
    out_shape=jax.ShapeDtypeStruct((N_NODES, D), jnp.float32),
)


def _tc_body(part_ref, cnt_ref, x_ref, xr_ref, wl_ref, o_ref):
    p = part_ref[0] + part_ref[1]
    gsum = p[:N_NODES]
    cnt8 = jnp.sum(cnt_ref[...], axis=0)
    rsel = (lax.broadcasted_iota(jnp.int32, (ACC_N, CH), 0) >> 7
            == lax.broadcasted_iota(jnp.int32, (ACC_N, CH), 1))
    lsel = ((lax.broadcasted_iota(jnp.int32, (ACC_N, D), 0) & (D - 1))
            == lax.broadcasted_iota(jnp.int32, (ACC_N, D), 1))
    cnt_rows = lax.dot_general(rsel.astype(jnp.float32), cnt8,
                               (((1,), (0,)), ((), ())),
                               preferred_element_type=jnp.float32)
    cnt = jnp.sum(cnt_rows * lsel.astype(jnp.float32), axis=1,
                  keepdims=True)[:N_NODES]
    xb = x_ref[...]
    agg = gsum / jnp.maximum(cnt, 1.0) - xb * (cnt > 0.0).astype(jnp.float32)
    o_ref[...] = (
        lax.dot_general(agg, wl_ref[...], (((1,), (1,)), ((), ())),
                        preferred_element_type=jnp.float32)
        + xr_ref[...]
    )


_tc_finish = pl.pallas_call(
    _tc_body,
    out_shape=jax.ShapeDtypeStruct((N_NODES, D), jnp.float32),
)


def kernel(x, edge_index, W_l, b_l, W_r):
    er = edge_index.reshape(-1)
    part, cnts = _sc_gather_scatter(x, er)
    xr = _tc_pre(x, b_l.reshape(1, D), W_r)
    return _tc_finish(part, cnts.reshape(NW, CH, D), x, xr, W_l)
